# baseline (speedup 1.0000x reference)
import functools

import jax
import jax.numpy as jnp
from jax import lax
from jax.experimental import pallas as pl
from jax.experimental.pallas import tpu as pltpu
from jax.experimental.pallas import tpu_sc as plsc

HIDDEN = 128
EPS = 1e-12

NUM_CORES = 2
NUM_SUBCORES = 16
NW = NUM_CORES * NUM_SUBCORES
C = 64
NBUF = 4
LANES = 16
VPH = HIDDEN // LANES
TGROUP = 16


def _permute(v, idx2d):
    return lax.gather(
        v, idx2d,
        lax.GatherDimensionNumbers(
            offset_dims=(), collapsed_slice_dims=(0,), start_index_map=(0,)),
        (1,), mode=lax.GatherScatterMode.PROMISE_IN_BOUNDS)


def _rsqrt_vec(x):
    i = lax.bitcast_convert_type(x, jnp.int32)
    i = jnp.int32(0x5F3759DF) - (i >> 1)
    y = lax.bitcast_convert_type(i, jnp.float32)
    half = x * jnp.float32(0.5)
    for _ in range(2):
        y = y * (jnp.float32(1.5) - half * y * y)
    return y


def _sc_body(ids_w2, ids_p2, ids_t2, wtab, ptab, ttab, gamma, beta, out,
             idsw_v, idsp_v, idst_v, wb, pb, ob, g_v, b_v, t_v,
             semg, semwb, *, n_tokens):
    wid = lax.axis_index("s") * NUM_CORES + lax.axis_index("c")
    per_w = n_tokens // NW
    ch = per_w // C
    base0 = wid * per_w

    pltpu.sync_copy(gamma, g_v)
    pltpu.sync_copy(beta, b_v)
    pltpu.sync_copy(ttab, t_v)
    pltpu.sync_copy(ids_w2.at[wid], idsw_v)
    pltpu.sync_copy(ids_p2.at[wid], idsp_v)
    pltpu.sync_copy(ids_t2.at[wid], idst_v)

    inv_h = jnp.float32(1.0 / HIDDEN)
    lane = lax.iota(jnp.int32, LANES)
    perms = [(lane ^ (1 << k)).reshape(LANES, 1) for k in range(4)]
    splat_idx = [jnp.full((LANES, 1), u, jnp.int32) for u in range(TGROUP)]
    t0s = [t_v[pl.ds(j * LANES, LANES)] for j in range(VPH)]
    tds = [t_v[pl.ds(HIDDEN + j * LANES, LANES)] - t0s[j] for j in range(VPH)]

    def fire(g, par):
        sl = pl.ds(g * C, C)
        pltpu.async_copy(wtab.at[idsw_v.at[sl]], wb.at[par], semg.at[par])
        pltpu.async_copy(ptab.at[idsp_v.at[sl]], pb.at[par], semg.at[par])

    def drain(g, par):
        sl = pl.ds(g * C, C)
        pltpu.make_async_copy(wtab.at[idsw_v.at[sl]], wb.at[par],
                              semg.at[par]).wait()
        pltpu.make_async_copy(ptab.at[idsp_v.at[sl]], pb.at[par],
                              semg.at[par]).wait()

    def out_slice(g):
        return out.at[pl.ds(base0 + g * C, C)]

    def compute(g, par):
        def tok16(i16, _):
            tb = i16 * TGROUP
            tv16 = idst_v[pl.ds(g * C + tb, TGROUP)]
            for u in range(TGROUP):
                t = tb + u
                tidf = _permute(tv16, splat_idx[u]).astype(jnp.float32)
                accs = []
                vsum = jnp.zeros((LANES,), jnp.float32)
                vsq = jnp.zeros((LANES,), jnp.float32)
                for j in range(VPH):
                    sl = pl.ds(j * LANES, LANES)
                    a = (wb[par, t, sl] + pb[par, t, sl]
                         + (t0s[j] + tidf * tds[j]))
                    accs.append(a)
                    vsum = vsum + a
                    vsq = vsq + a * a
                for p in perms:
                    vsum = vsum + _permute(vsum, p)
                    vsq = vsq + _permute(vsq, p)
                mv = vsum * inv_h
                var = vsq * inv_h - mv * mv
                rstd = _rsqrt_vec(var + jnp.float32(EPS))
                for j in range(VPH):
                    sl = pl.ds(j * LANES, LANES)
                    ob[par, t, sl] = (accs[j] - mv) * rstd * g_v[sl] + b_v[sl]
            return _

        lax.fori_loop(0, C // TGROUP, tok16, None)

    for g0 in range(NBUF - 1):
        fire(g0, g0)

    def pipe(i, _):
        for s in range(NBUF):
            g = i * NBUF + s
            nx = g + NBUF - 1

            @pl.when(nx < ch)
            def _(nx=nx, s=s):
                fire(nx, (s + NBUF - 1) % NBUF)

            drain(g, s)

            @pl.when(g >= NBUF)
            def _(g=g, s=s):
                pltpu.make_async_copy(ob.at[s], out_slice(g - NBUF),
                                      semwb.at[s]).wait()

            compute(g, s)
            pltpu.async_copy(ob.at[s], out_slice(g), semwb.at[s])
        return _

    lax.fori_loop(0, ch // NBUF, pipe, None)

    for g in range(ch - NBUF, ch):
        pltpu.make_async_copy(ob.at[g % NBUF], out_slice(g),
                              semwb.at[g % NBUF]).wait()


def kernel(input_ids, token_type_ids, position_ids, word_emb, pos_emb,
           type_emb, ln_gamma, ln_beta):
    B, L = input_ids.shape
    n = B * L
    per_w = n // NW
    ids_w = input_ids.reshape(NW, per_w).astype(jnp.int32)
    ids_t = token_type_ids.reshape(NW, per_w).astype(jnp.int32)
    ids_p = position_ids.reshape(NW, per_w).astype(jnp.int32)

    mesh = plsc.VectorSubcoreMesh(
        core_axis_name="c", subcore_axis_name="s",
        num_cores=NUM_CORES, num_subcores=NUM_SUBCORES)

    run = pl.kernel(
        functools.partial(_sc_body, n_tokens=n),
        out_type=jax.ShapeDtypeStruct((n, HIDDEN), jnp.float32),
        mesh=mesh,
        scratch_types=[
            pltpu.VMEM((per_w,), jnp.int32),
            pltpu.VMEM((per_w,), jnp.int32),
            pltpu.VMEM((per_w,), jnp.int32),
            pltpu.VMEM((NBUF, C, HIDDEN), jnp.float32),
            pltpu.VMEM((NBUF, C, HIDDEN), jnp.float32),
            pltpu.VMEM((NBUF, C, HIDDEN), jnp.float32),
            pltpu.VMEM((HIDDEN,), jnp.float32),
            pltpu.VMEM((HIDDEN,), jnp.float32),
            pltpu.VMEM((2 * HIDDEN,), jnp.float32),
            pltpu.SemaphoreType.DMA((NBUF,)),
            pltpu.SemaphoreType.DMA((NBUF,)),
        ],
    )
    out = run(ids_w, ids_p, ids_t, word_emb, pos_emb,
              type_emb.reshape(-1), ln_gamma, ln_beta)
    return out.reshape(B, L, HIDDEN)


# speedup vs baseline: 1.4269x; 1.4269x over previous
import functools

import jax
import jax.numpy as jnp
from jax import lax
from jax.experimental import pallas as pl
from jax.experimental.pallas import tpu as pltpu
from jax.experimental.pallas import tpu_sc as plsc

HIDDEN = 128
EPS = 1e-12

NUM_CORES = 2
NUM_SUBCORES = 16
NW = NUM_CORES * NUM_SUBCORES
C = 128
NBUF = 2
LANES = 16
VPH = HIDDEN // LANES
TGROUP = 16


def _permute(v, idx2d):
    return lax.gather(
        v, idx2d,
        lax.GatherDimensionNumbers(
            offset_dims=(), collapsed_slice_dims=(0,), start_index_map=(0,)),
        (1,), mode=lax.GatherScatterMode.PROMISE_IN_BOUNDS)


def _rsqrt_vec(x):
    i = lax.bitcast_convert_type(x, jnp.int32)
    i = jnp.int32(0x5F3759DF) - (i >> 1)
    y = lax.bitcast_convert_type(i, jnp.float32)
    half = x * jnp.float32(0.5)
    for _ in range(2):
        y = y * (jnp.float32(1.5) - half * y * y)
    return y


def _sc_body(ids_w2, ids_c2, wtab, ctab, gamma, beta, out,
             idsw_v, idsc_v, wb, pb, ob, g_v, b_v,
             semg, semwb, *, n_tokens):
    wid = lax.axis_index("s") * NUM_CORES + lax.axis_index("c")
    per_w = n_tokens // NW
    ch = per_w // C
    base0 = wid * per_w

    pltpu.sync_copy(gamma, g_v)
    pltpu.sync_copy(beta, b_v)
    pltpu.sync_copy(ids_w2.at[wid], idsw_v)
    pltpu.sync_copy(ids_c2.at[wid], idsc_v)

    inv_h = jnp.float32(1.0 / HIDDEN)
    lane = lax.iota(jnp.int32, LANES)
    perms = [(lane ^ (1 << k)).reshape(LANES, 1) for k in range(4)]

    def fire(g, par):
        sl = pl.ds(g * C, C)
        pltpu.async_copy(wtab.at[idsw_v.at[sl]], wb.at[par], semg.at[par])
        pltpu.async_copy(ctab.at[idsc_v.at[sl]], pb.at[par], semg.at[par])

    def drain(g, par):
        sl = pl.ds(g * C, C)
        pltpu.make_async_copy(wtab.at[idsw_v.at[sl]], wb.at[par],
                              semg.at[par]).wait()
        pltpu.make_async_copy(ctab.at[idsc_v.at[sl]], pb.at[par],
                              semg.at[par]).wait()

    def out_slice(g):
        return out.at[pl.ds(base0 + g * C, C)]

    def compute(g, par):
        def tok16(i16, _):
            tb = i16 * TGROUP
            for u in range(TGROUP):
                t = tb + u
                accs = []
                vsum = jnp.zeros((LANES,), jnp.float32)
                vsq = jnp.zeros((LANES,), jnp.float32)
                for j in range(VPH):
                    sl = pl.ds(j * LANES, LANES)
                    a = wb[par, t, sl] + pb[par, t, sl]
                    accs.append(a)
                    vsum = vsum + a
                    vsq = vsq + a * a
                for p in perms:
                    vsum = vsum + _permute(vsum, p)
                    vsq = vsq + _permute(vsq, p)
                mv = vsum * inv_h
                var = vsq * inv_h - mv * mv
                rstd = _rsqrt_vec(var + jnp.float32(EPS))
                for j in range(VPH):
                    sl = pl.ds(j * LANES, LANES)
                    ob[par, t, sl] = (accs[j] - mv) * rstd * g_v[sl] + b_v[sl]
            return _

        lax.fori_loop(0, C // TGROUP, tok16, None)

    for g0 in range(NBUF - 1):
        fire(g0, g0)

    def pipe(i, _):
        for s in range(NBUF):
            g = i * NBUF + s
            nx = g + NBUF - 1

            @pl.when(nx < ch)
            def _(nx=nx, s=s):
                fire(nx, (s + NBUF - 1) % NBUF)

            drain(g, s)

            @pl.when(g >= NBUF)
            def _(g=g, s=s):
                pltpu.make_async_copy(ob.at[s], out_slice(g - NBUF),
                                      semwb.at[s]).wait()

            compute(g, s)
            pltpu.async_copy(ob.at[s], out_slice(g), semwb.at[s])
        return _

    lax.fori_loop(0, ch // NBUF, pipe, None)

    for g in range(ch - NBUF, ch):
        pltpu.make_async_copy(ob.at[g % NBUF], out_slice(g),
                              semwb.at[g % NBUF]).wait()


def kernel(input_ids, token_type_ids, position_ids, word_emb, pos_emb,
           type_emb, ln_gamma, ln_beta):
    B, L = input_ids.shape
    n = B * L
    per_w = n // NW
    tv = type_emb.shape[0]
    ids_w = input_ids.reshape(NW, per_w).astype(jnp.int32)
    ids_c = (position_ids.astype(jnp.int32) * tv
             + token_type_ids.astype(jnp.int32)).reshape(NW, per_w)
    ctab = (pos_emb[:, None, :] + type_emb[None, :, :]).reshape(-1, HIDDEN)

    mesh = plsc.VectorSubcoreMesh(
        core_axis_name="c", subcore_axis_name="s",
        num_cores=NUM_CORES, num_subcores=NUM_SUBCORES)

    run = pl.kernel(
        functools.partial(_sc_body, n_tokens=n),
        out_type=jax.ShapeDtypeStruct((n, HIDDEN), jnp.float32),
        mesh=mesh,
        scratch_types=[
            pltpu.VMEM((per_w,), jnp.int32),
            pltpu.VMEM((per_w,), jnp.int32),
            pltpu.VMEM((NBUF, C, HIDDEN), jnp.float32),
            pltpu.VMEM((NBUF, C, HIDDEN), jnp.float32),
            pltpu.VMEM((NBUF, C, HIDDEN), jnp.float32),
            pltpu.VMEM((HIDDEN,), jnp.float32),
            pltpu.VMEM((HIDDEN,), jnp.float32),
            pltpu.SemaphoreType.DMA((NBUF,)),
            pltpu.SemaphoreType.DMA((NBUF,)),
        ],
    )
    out = run(ids_w, ids_c, word_emb, ctab, ln_gamma, ln_beta)
    return out.reshape(B, L, HIDDEN)
